# Initial kernel scaffold; baseline (speedup 1.0000x reference)
import jax
import jax.numpy as jnp
from jax.experimental import pallas as pl


def kernel(x, edge_index, W, b, gamma, beta, a):
    raise NotImplementedError("write your pallas kernel here")


# speedup vs baseline: 18.0570x; 18.0570x over previous
import functools

import jax
import jax.numpy as jnp
from jax import lax
from jax.experimental import pallas as pl
from jax.experimental.pallas import tpu as pltpu
from jax.experimental.pallas import tpu_sc as plsc

_NC = 2
_NS = 16
_NW = _NC * _NS
_CH = 80


def _sc_mesh():
    return plsc.VectorSubcoreMesh(core_axis_name="c", subcore_axis_name="s")


def _make_deg_kernel(n, e):
    ept = e // _NW
    nch = ept // _CH
    zch = 624
    tail = n - _NS * zch

    @functools.partial(
        pl.kernel,
        out_type=jax.ShapeDtypeStruct((_NC * n,), jnp.float32),
        mesh=_sc_mesh(),
        scratch_types=[
            pltpu.VMEM((_CH,), jnp.int32),
            pltpu.VMEM((_CH,), jnp.float32),
            pltpu.VMEM((zch,), jnp.float32),
            pltpu.VMEM_SHARED((n,), jnp.float32),
        ],
    )
    def deg_kernel(col_hbm, ones_hbm, zeros_hbm, out_hbm,
                   colbuf, onesbuf, fbuf, hist):
        c = lax.axis_index("c")
        s = lax.axis_index("s")
        w = c * _NS + s

        pltpu.sync_copy(zeros_hbm, fbuf)
        pltpu.sync_copy(fbuf, hist.at[pl.ds(s * zch, zch)])

        @pl.when(s == 0)
        def _():
            pltpu.sync_copy(fbuf.at[pl.ds(0, tail)],
                            hist.at[pl.ds(_NS * zch, tail)])

        pltpu.sync_copy(ones_hbm, onesbuf)
        plsc.subcore_barrier()

        @pl.loop(0, nch)
        def _(i):
            off = w * ept + i * _CH
            pltpu.sync_copy(col_hbm.at[pl.ds(off, _CH)], colbuf)
            pltpu.sync_copy(onesbuf, hist.at[colbuf], add=True)

        plsc.subcore_barrier()
        pltpu.sync_copy(hist.at[pl.ds(s * zch, zch)], fbuf)
        pltpu.sync_copy(fbuf, out_hbm.at[pl.ds(c * n + s * zch, zch)])

        @pl.when(s == 0)
        def _():
            pltpu.sync_copy(hist.at[pl.ds(_NS * zch, tail)],
                            fbuf.at[pl.ds(0, tail)])
            pltpu.sync_copy(fbuf.at[pl.ds(0, tail)],
                            out_hbm.at[pl.ds(c * n + _NS * zch, tail)])

    return deg_kernel


def _make_scatter_kernel(n, d, e):
    ept = e // _NW
    nch = ept // _CH
    rpt = 624
    tail = n - _NS * rpt
    nz = rpt // _CH
    rz = rpt - nz * _CH

    @functools.partial(
        pl.kernel,
        out_type=jax.ShapeDtypeStruct((_NC, n, d), jnp.float32),
        mesh=_sc_mesh(),
        scratch_types=[
            pltpu.VMEM((_CH,), jnp.int32),
            pltpu.VMEM((_CH,), jnp.int32),
            pltpu.VMEM((_CH, d), jnp.float32),
            pltpu.VMEM_SHARED((n, d), jnp.float32),
            pltpu.SemaphoreType.DMA,
        ],
    )
    def scatter_kernel(row_hbm, col_hbm, y_hbm, zeros_hbm, out_hbm,
                       rowbuf, colbuf, gbuf, acc, sem):
        c = lax.axis_index("c")
        s = lax.axis_index("s")
        w = c * _NS + s
        r0 = s * rpt

        pltpu.sync_copy(zeros_hbm, gbuf)
        for j in range(nz):
            pltpu.sync_copy(gbuf, acc.at[pl.ds(r0 + j * _CH, _CH)])
        if rz:
            pltpu.sync_copy(gbuf.at[pl.ds(0, rz)],
                            acc.at[pl.ds(r0 + nz * _CH, rz)])

        @pl.when(s == 0)
        def _():
            pltpu.sync_copy(gbuf.at[pl.ds(0, tail)],
                            acc.at[pl.ds(_NS * rpt, tail)])

        plsc.subcore_barrier()

        @pl.loop(0, nch)
        def _(i):
            off = w * ept + i * _CH
            pltpu.sync_copy(row_hbm.at[pl.ds(off, _CH)], rowbuf)
            pltpu.sync_copy(col_hbm.at[pl.ds(off, _CH)], colbuf)
            pltpu.async_copy(y_hbm.at[rowbuf], gbuf, sem).wait()
            pltpu.sync_copy(gbuf, acc.at[colbuf], add=True)

        plsc.subcore_barrier()
        for j in range(nz):
            pltpu.sync_copy(acc.at[pl.ds(r0 + j * _CH, _CH)], gbuf)
            pltpu.sync_copy(gbuf, out_hbm.at[c, pl.ds(r0 + j * _CH, _CH)])
        if rz:
            pltpu.sync_copy(acc.at[pl.ds(r0 + nz * _CH, rz)],
                            gbuf.at[pl.ds(0, rz)])
            pltpu.sync_copy(gbuf.at[pl.ds(0, rz)],
                            out_hbm.at[c, pl.ds(r0 + nz * _CH, rz)])

        @pl.when(s == 0)
        def _():
            pltpu.sync_copy(acc.at[pl.ds(_NS * rpt, tail)],
                            gbuf.at[pl.ds(0, tail)])
            pltpu.sync_copy(gbuf.at[pl.ds(0, tail)],
                            out_hbm.at[c, pl.ds(_NS * rpt, tail)])

    return scatter_kernel


def _mm_body(x_ref, w_ref, p_ref, y_ref):
    deg = 1.0 + p_ref[0, :] + p_ref[1, :]
    dinv = lax.rsqrt(deg)
    xw = jnp.dot(x_ref[...], w_ref[...], preferred_element_type=jnp.float32)
    y_ref[...] = xw * dinv[:, None]


def _final_body(s_ref, y_ref, p_ref, gamma_ref, beta_ref, a_ref, o_ref):
    deg = 1.0 + p_ref[0, :] + p_ref[1, :]
    dinv = lax.rsqrt(deg)
    h = (s_ref[0] + s_ref[1] + y_ref[...]) * dinv[:, None]
    mean = jnp.mean(h, axis=0)
    ctr = h - mean[None, :]
    var = jnp.mean(ctr * ctr, axis=0)
    hn = ctr * lax.rsqrt(var + 1e-5) * gamma_ref[...] + beta_ref[...]
    av = a_ref[...]
    o_ref[...] = jnp.where(hn >= 0, hn, av * hn)


def kernel(x, edge_index, W, b, gamma, beta, a):
    n, d = x.shape
    e = edge_index.shape[1]
    row = edge_index[0]
    col = edge_index[1]

    ones_ch = jnp.ones((_CH,), jnp.float32)
    zeros_z = jnp.zeros((624,), jnp.float32)
    zeros_chd = jnp.zeros((_CH, d), jnp.float32)

    p = _make_deg_kernel(n, e)(col, ones_ch, zeros_z).reshape(_NC, n)
    y = pl.pallas_call(
        _mm_body,
        out_shape=jax.ShapeDtypeStruct((n, d), jnp.float32),
    )(x, W, p)
    s = _make_scatter_kernel(n, d, e)(row, col, y, zeros_chd)
    out = pl.pallas_call(
        _final_body,
        out_shape=jax.ShapeDtypeStruct((n, d), jnp.float32),
    )(s, y, p, gamma, beta, a)
    return out


# speedup vs baseline: 45.7979x; 2.5363x over previous
import functools

import jax
import jax.numpy as jnp
from jax import lax
from jax.experimental import pallas as pl
from jax.experimental.pallas import tpu as pltpu
from jax.experimental.pallas import tpu_sc as plsc

_NC = 2
_NS = 16
_NW = _NC * _NS
_CH = 125
_ZCH = 104
_RPT = 624


def _sc_mesh():
    return plsc.VectorSubcoreMesh(core_axis_name="c", subcore_axis_name="s")


def _make_deg_kernel(n, e):
    nch = e // (_NW * _CH)
    zch = _RPT
    tail = n - _NS * zch
    grp = 16

    @functools.partial(
        pl.kernel,
        out_type=jax.ShapeDtypeStruct((_NC * n,), jnp.float32),
        mesh=_sc_mesh(),
        scratch_types=[
            pltpu.VMEM((nch, _CH), jnp.int32),
            pltpu.VMEM((_CH,), jnp.float32),
            pltpu.VMEM((zch,), jnp.float32),
            pltpu.VMEM_SHARED((n,), jnp.float32),
            pltpu.SemaphoreType.DMA,
        ],
    )
    def deg_kernel(col_hbm, ones_hbm, zeros_hbm, out_hbm,
                   colbig, onesbuf, fbuf, hist, sem):
        c = lax.axis_index("c")
        s = lax.axis_index("s")
        w = c * _NS + s

        pltpu.sync_copy(zeros_hbm, fbuf)
        pltpu.sync_copy(fbuf, hist.at[pl.ds(s * zch, zch)])

        @pl.when(s == 0)
        def _():
            pltpu.sync_copy(fbuf.at[pl.ds(0, tail)],
                            hist.at[pl.ds(_NS * zch, tail)])

        pltpu.sync_copy(ones_hbm, onesbuf)
        pltpu.sync_copy(col_hbm.at[w], colbig)
        plsc.subcore_barrier()

        def issue(j):
            pltpu.async_copy(onesbuf, hist.at[colbig.at[j]], sem, add=True)

        def drain(j):
            pltpu.make_async_copy(onesbuf, hist.at[colbig.at[j]], sem).wait()

        for j in range(grp):
            issue(j)

        @pl.loop(0, nch // grp - 1)
        def _(t):
            for k in range(grp):
                issue((t + 1) * grp + k)
            for k in range(grp):
                drain(t * grp + k)

        for j in range(grp):
            drain(nch - grp + j)

        plsc.subcore_barrier()
        pltpu.sync_copy(hist.at[pl.ds(s * zch, zch)], fbuf)
        pltpu.sync_copy(fbuf, out_hbm.at[pl.ds(c * n + s * zch, zch)])

        @pl.when(s == 0)
        def _():
            pltpu.sync_copy(hist.at[pl.ds(_NS * zch, tail)],
                            fbuf.at[pl.ds(0, tail)])
            pltpu.sync_copy(fbuf.at[pl.ds(0, tail)],
                            out_hbm.at[pl.ds(c * n + _NS * zch, tail)])

    return deg_kernel


def _make_scatter_kernel(n, d, e):
    nch = e // (_NW * _CH)
    tail = n - _NS * _RPT
    nz = _RPT // _ZCH

    @functools.partial(
        pl.kernel,
        out_type=jax.ShapeDtypeStruct((_NC, n, d), jnp.float32),
        mesh=_sc_mesh(),
        scratch_types=[
            [pltpu.VMEM((_CH,), jnp.int32)] * 4,
            [pltpu.VMEM((_CH,), jnp.int32)] * 4,
            [pltpu.VMEM((_CH, d), jnp.float32)] * 2,
            [pltpu.SemaphoreType.DMA] * 4,
            [pltpu.SemaphoreType.DMA] * 2,
            [pltpu.SemaphoreType.DMA] * 2,
            pltpu.VMEM_SHARED((n, d), jnp.float32),
        ],
    )
    def scatter_kernel(row_hbm, col_hbm, y_hbm, zeros_hbm, out_hbm,
                       rowbuf, colbuf, gbuf, isem, gsem, ssem, acc):
        c = lax.axis_index("c")
        s = lax.axis_index("s")
        w = c * _NS + s
        r0 = s * _RPT

        pltpu.sync_copy(zeros_hbm, gbuf[0].at[pl.ds(0, _ZCH)])
        for j in range(nz):
            pltpu.sync_copy(gbuf[0].at[pl.ds(0, _ZCH)],
                            acc.at[pl.ds(r0 + j * _ZCH, _ZCH)])

        @pl.when(s == 0)
        def _():
            pltpu.sync_copy(gbuf[0].at[pl.ds(0, tail)],
                            acc.at[pl.ds(_NS * _RPT, tail)])

        plsc.subcore_barrier()

        def idx_issue(ch, i):
            pltpu.async_copy(row_hbm.at[w, ch], rowbuf[i], isem[i])
            pltpu.async_copy(col_hbm.at[w, ch], colbuf[i], isem[i])

        def idx_wait(ch, i):
            pltpu.make_async_copy(row_hbm.at[w, ch], rowbuf[i],
                                  isem[i]).wait()
            pltpu.make_async_copy(col_hbm.at[w, ch], colbuf[i],
                                  isem[i]).wait()

        def gather(i, b):
            pltpu.async_copy(y_hbm.at[rowbuf[i]], gbuf[b], gsem[b])

        def gather_wait(i, b):
            pltpu.make_async_copy(y_hbm.at[rowbuf[i]], gbuf[b],
                                  gsem[b]).wait()

        def scat(i, b):
            pltpu.async_copy(gbuf[b], acc.at[colbuf[i]], ssem[b], add=True)

        def scat_wait(i, b):
            pltpu.make_async_copy(gbuf[b], acc.at[colbuf[i]],
                                  ssem[b]).wait()

        def a_step(ch):
            if ch >= 2:
                scat_wait((ch - 2) % 4, (ch - 2) % 2)
            if ch + 2 < nch:
                idx_issue(ch + 2, (ch + 2) % 4)
            idx_wait(ch, ch % 4)
            gather(ch % 4, ch % 2)

        def b_step(ch):
            gather_wait(ch % 4, ch % 2)
            scat(ch % 4, ch % 2)

        idx_issue(0, 0)
        idx_issue(1, 1)
        a_step(0)
        a_step(1)
        b_step(0)
        a_step(2)
        b_step(1)
        a_step(3)
        b_step(2)
        a_step(4)
        b_step(3)
        a_step(5)
        b_step(4)

        @pl.loop(1, nch // 4 - 1)
        def _(t):
            for k in range(4):
                ch = 4 * t + k
                scat_wait(k % 4, k % 2)
                idx_issue(ch + 4, k % 4)
                idx_wait(ch + 2, (k + 2) % 4)
                gather((k + 2) % 4, k % 2)
                gather_wait((k + 1) % 4, (k + 1) % 2)
                scat((k + 1) % 4, (k + 1) % 2)

        a_step(nch - 2)
        b_step(nch - 3)
        a_step(nch - 1)
        b_step(nch - 2)
        b_step(nch - 1)
        scat_wait((nch - 2) % 4, (nch - 2) % 2)
        scat_wait((nch - 1) % 4, (nch - 1) % 2)

        plsc.subcore_barrier()
        for j in range(nz):
            pltpu.sync_copy(acc.at[pl.ds(r0 + j * _ZCH, _ZCH)],
                            gbuf[0].at[pl.ds(0, _ZCH)])
            pltpu.sync_copy(gbuf[0].at[pl.ds(0, _ZCH)],
                            out_hbm.at[c, pl.ds(r0 + j * _ZCH, _ZCH)])

        @pl.when(s == 0)
        def _():
            pltpu.sync_copy(acc.at[pl.ds(_NS * _RPT, tail)],
                            gbuf[1].at[pl.ds(0, tail)])
            pltpu.sync_copy(gbuf[1].at[pl.ds(0, tail)],
                            out_hbm.at[c, pl.ds(_NS * _RPT, tail)])

    return scatter_kernel


def _mm_body(x_ref, w_ref, p_ref, y_ref):
    deg = 1.0 + p_ref[0, :] + p_ref[1, :]
    dinv = lax.rsqrt(deg)
    xw = jnp.dot(x_ref[...], w_ref[...], preferred_element_type=jnp.float32)
    y_ref[...] = xw * dinv[:, None]


def _final_body(s_ref, y_ref, p_ref, gamma_ref, beta_ref, a_ref, o_ref):
    deg = 1.0 + p_ref[0, :] + p_ref[1, :]
    dinv = lax.rsqrt(deg)
    h = (s_ref[0] + s_ref[1] + y_ref[...]) * dinv[:, None]
    mean = jnp.mean(h, axis=0)
    ctr = h - mean[None, :]
    var = jnp.mean(ctr * ctr, axis=0)
    hn = ctr * lax.rsqrt(var + 1e-5) * gamma_ref[...] + beta_ref[...]
    av = a_ref[...]
    o_ref[...] = jnp.where(hn >= 0, hn, av * hn)


def kernel(x, edge_index, W, b, gamma, beta, a):
    n, d = x.shape
    e = edge_index.shape[1]
    row3 = edge_index[0].reshape(_NW, e // (_NW * _CH), _CH)
    col3 = edge_index[1].reshape(_NW, e // (_NW * _CH), _CH)

    ones_ch = jnp.ones((_CH,), jnp.float32)
    zeros_z = jnp.zeros((_RPT,), jnp.float32)
    zeros_chd = jnp.zeros((_ZCH, d), jnp.float32)

    p = _make_deg_kernel(n, e)(col3, ones_ch, zeros_z).reshape(_NC, n)
    y = pl.pallas_call(
        _mm_body,
        out_shape=jax.ShapeDtypeStruct((n, d), jnp.float32),
    )(x, W, p)
    s = _make_scatter_kernel(n, d, e)(row3, col3, y, zeros_chd)
    out = pl.pallas_call(
        _final_body,
        out_shape=jax.ShapeDtypeStruct((n, d), jnp.float32),
    )(s, y, p, gamma, beta, a)
    return out


# speedup vs baseline: 46.0169x; 1.0048x over previous
import functools

import jax
import jax.numpy as jnp
from jax import lax
from jax.experimental import pallas as pl
from jax.experimental.pallas import tpu as pltpu
from jax.experimental.pallas import tpu_sc as plsc

_NC = 2
_NS = 16
_NW = _NC * _NS
_CH = 125
_SCH = 80
_ZCH = 48
_RPT = 624


def _sc_mesh():
    return plsc.VectorSubcoreMesh(core_axis_name="c", subcore_axis_name="s")


def _make_deg_kernel(n, e):
    nch = e // (_NW * _CH)
    zch = _RPT
    tail = n - _NS * zch
    grp = 16

    @functools.partial(
        pl.kernel,
        out_type=jax.ShapeDtypeStruct((_NC * n,), jnp.float32),
        mesh=_sc_mesh(),
        scratch_types=[
            pltpu.VMEM((nch, _CH), jnp.int32),
            pltpu.VMEM((_CH,), jnp.float32),
            pltpu.VMEM((zch,), jnp.float32),
            pltpu.VMEM_SHARED((n,), jnp.float32),
            pltpu.SemaphoreType.DMA,
        ],
    )
    def deg_kernel(col_hbm, ones_hbm, zeros_hbm, out_hbm,
                   colbig, onesbuf, fbuf, hist, sem):
        c = lax.axis_index("c")
        s = lax.axis_index("s")
        w = c * _NS + s

        pltpu.sync_copy(zeros_hbm, fbuf)
        pltpu.sync_copy(fbuf, hist.at[pl.ds(s * zch, zch)])

        @pl.when(s == 0)
        def _():
            pltpu.sync_copy(fbuf.at[pl.ds(0, tail)],
                            hist.at[pl.ds(_NS * zch, tail)])

        pltpu.sync_copy(ones_hbm, onesbuf)
        pltpu.sync_copy(col_hbm.at[w], colbig)
        plsc.subcore_barrier()

        def issue(j):
            pltpu.async_copy(onesbuf, hist.at[colbig.at[j]], sem, add=True)

        def drain(j):
            pltpu.make_async_copy(onesbuf, hist.at[colbig.at[j]], sem).wait()

        for j in range(grp):
            issue(j)

        @pl.loop(0, nch // grp - 1)
        def _(t):
            for k in range(grp):
                issue((t + 1) * grp + k)
            for k in range(grp):
                drain(t * grp + k)

        for j in range(grp):
            drain(nch - grp + j)

        plsc.subcore_barrier()
        pltpu.sync_copy(hist.at[pl.ds(s * zch, zch)], fbuf)
        pltpu.sync_copy(fbuf, out_hbm.at[pl.ds(c * n + s * zch, zch)])

        @pl.when(s == 0)
        def _():
            pltpu.sync_copy(hist.at[pl.ds(_NS * zch, tail)],
                            fbuf.at[pl.ds(0, tail)])
            pltpu.sync_copy(fbuf.at[pl.ds(0, tail)],
                            out_hbm.at[pl.ds(c * n + _NS * zch, tail)])

    return deg_kernel


def _make_scatter_kernel(n, d, e):
    nch = e // (_NW * _SCH)
    tail = n - _NS * _RPT
    nz = _RPT // _ZCH

    @functools.partial(
        pl.kernel,
        out_type=jax.ShapeDtypeStruct((_NC, n, d), jnp.float32),
        mesh=_sc_mesh(),
        scratch_types=[
            [pltpu.VMEM((_SCH,), jnp.int32)] * 8,
            [pltpu.VMEM((_SCH,), jnp.int32)] * 8,
            [pltpu.VMEM((_SCH, d), jnp.float32)] * 4,
            [pltpu.SemaphoreType.DMA] * 8,
            [pltpu.SemaphoreType.DMA] * 4,
            [pltpu.SemaphoreType.DMA] * 4,
            pltpu.VMEM_SHARED((n, d), jnp.float32),
        ],
    )
    def scatter_kernel(row_hbm, col_hbm, y_hbm, zeros_hbm, out_hbm,
                       rowbuf, colbuf, gbuf, isem, gsem, ssem, acc):
        c = lax.axis_index("c")
        s = lax.axis_index("s")
        w = c * _NS + s
        e0 = w * (e // _NW)
        r0 = s * _RPT

        pltpu.sync_copy(zeros_hbm, gbuf[0].at[pl.ds(0, _ZCH)])
        for j in range(nz):
            pltpu.sync_copy(gbuf[0].at[pl.ds(0, _ZCH)],
                            acc.at[pl.ds(r0 + j * _ZCH, _ZCH)])

        @pl.when(s == 0)
        def _():
            pltpu.sync_copy(gbuf[0].at[pl.ds(0, tail)],
                            acc.at[pl.ds(_NS * _RPT, tail)])

        plsc.subcore_barrier()

        def idx_issue(ch, i):
            sl = pl.ds(e0 + ch * _SCH, _SCH)
            pltpu.async_copy(row_hbm.at[sl], rowbuf[i], isem[i])
            pltpu.async_copy(col_hbm.at[sl], colbuf[i], isem[i])

        def idx_wait(ch, i):
            sl = pl.ds(e0 + ch * _SCH, _SCH)
            pltpu.make_async_copy(row_hbm.at[sl], rowbuf[i],
                                  isem[i]).wait()
            pltpu.make_async_copy(col_hbm.at[sl], colbuf[i],
                                  isem[i]).wait()

        def gather(i, b):
            pltpu.async_copy(y_hbm.at[rowbuf[i]], gbuf[b], gsem[b])

        def gather_wait(i, b):
            pltpu.make_async_copy(y_hbm.at[rowbuf[i]], gbuf[b],
                                  gsem[b]).wait()

        def scat(i, b):
            pltpu.async_copy(gbuf[b], acc.at[colbuf[i]], ssem[b], add=True)

        def scat_wait(i, b):
            pltpu.make_async_copy(gbuf[b], acc.at[colbuf[i]],
                                  ssem[b]).wait()

        def a_step(x):
            if x >= 4:
                scat_wait((x - 4) % 8, (x - 4) % 4)
            if x + 4 < nch:
                idx_issue(x + 4, (x + 4) % 8)
            idx_wait(x, x % 8)
            gather(x % 8, x % 4)

        def b_step(x):
            gather_wait(x % 8, x % 4)
            scat(x % 8, x % 4)

        for i in range(4):
            idx_issue(i, i)
        a_step(0)
        a_step(1)
        for cc in range(8):
            a_step(cc + 2)
            b_step(cc)

        t_end = (nch - 14) // 8 + 1

        @pl.loop(1, t_end)
        def _(t):
            for k in range(8):
                ch = 8 * t + k
                scat_wait((k + 6) % 8, (k + 2) % 4)
                idx_issue(ch + 6, (k + 6) % 8)
                idx_wait(ch + 2, (k + 2) % 8)
                gather((k + 2) % 8, (k + 2) % 4)
                gather_wait(k, k % 4)
                scat(k, k % 4)

        for cc in range(8 * t_end, nch - 2):
            a_step(cc + 2)
            b_step(cc)
        b_step(nch - 2)
        b_step(nch - 1)
        for x in range(nch - 4, nch):
            scat_wait(x % 8, x % 4)

        plsc.subcore_barrier()
        for j in range(nz):
            pltpu.sync_copy(acc.at[pl.ds(r0 + j * _ZCH, _ZCH)],
                            gbuf[0].at[pl.ds(0, _ZCH)])
            pltpu.sync_copy(gbuf[0].at[pl.ds(0, _ZCH)],
                            out_hbm.at[c, pl.ds(r0 + j * _ZCH, _ZCH)])

        @pl.when(s == 0)
        def _():
            pltpu.sync_copy(acc.at[pl.ds(_NS * _RPT, tail)],
                            gbuf[1].at[pl.ds(0, tail)])
            pltpu.sync_copy(gbuf[1].at[pl.ds(0, tail)],
                            out_hbm.at[c, pl.ds(_NS * _RPT, tail)])

    return scatter_kernel


def _mm_body(x_ref, w_ref, p_ref, y_ref):
    deg = 1.0 + p_ref[0, :] + p_ref[1, :]
    dinv = lax.rsqrt(deg)
    xw = jnp.dot(x_ref[...], w_ref[...], preferred_element_type=jnp.float32)
    y_ref[...] = xw * dinv[:, None]


def _final_body(s_ref, y_ref, p_ref, gamma_ref, beta_ref, a_ref, o_ref):
    deg = 1.0 + p_ref[0, :] + p_ref[1, :]
    dinv = lax.rsqrt(deg)
    h = (s_ref[0] + s_ref[1] + y_ref[...]) * dinv[:, None]
    mean = jnp.mean(h, axis=0)
    ctr = h - mean[None, :]
    var = jnp.mean(ctr * ctr, axis=0)
    hn = ctr * lax.rsqrt(var + 1e-5) * gamma_ref[...] + beta_ref[...]
    av = a_ref[...]
    o_ref[...] = jnp.where(hn >= 0, hn, av * hn)


def kernel(x, edge_index, W, b, gamma, beta, a):
    n, d = x.shape
    e = edge_index.shape[1]
    col3d = edge_index[1].reshape(_NW, e // (_NW * _CH), _CH)
    row1 = edge_index[0]
    col1 = edge_index[1]

    ones_ch = jnp.ones((_CH,), jnp.float32)
    zeros_z = jnp.zeros((_RPT,), jnp.float32)
    zeros_chd = jnp.zeros((_ZCH, d), jnp.float32)

    p = _make_deg_kernel(n, e)(col3d, ones_ch, zeros_z).reshape(_NC, n)
    y = pl.pallas_call(
        _mm_body,
        out_shape=jax.ShapeDtypeStruct((n, d), jnp.float32),
    )(x, W, p)
    s = _make_scatter_kernel(n, d, e)(row1, col1, y, zeros_chd)
    out = pl.pallas_call(
        _final_body,
        out_shape=jax.ShapeDtypeStruct((n, d), jnp.float32),
    )(s, y, p, gamma, beta, a)
    return out
